# baseline (speedup 1.0000x reference)
import functools

import jax
import jax.numpy as jnp
from jax import lax
from jax.experimental import pallas as pl
from jax.experimental.pallas import tpu as pltpu
from jax.experimental.pallas import tpu_sc as plsc


def kernel(x, pos_emb):
    B, S = x.shape
    D = pos_emb.shape[1]

    info = plsc.get_sparse_core_info()
    NC, NS = info.num_cores, info.num_subcores
    NW = NC * NS
    rows_per_w = S // NW

    mesh = plsc.VectorSubcoreMesh(core_axis_name="c", subcore_axis_name="s")

    @functools.partial(
        pl.kernel,
        out_type=jax.ShapeDtypeStruct((B, S, D), jnp.float32),
        mesh=mesh,
        scratch_types=[
            pltpu.SemaphoreType.DMA,
        ],
    )
    def body(pos_hbm, out_hbm, sem):
        wid = lax.axis_index("s") * NC + lax.axis_index("c")
        base = wid * rows_per_w
        cps = []
        for b in range(B):
            cp = pltpu.make_async_copy(
                pos_hbm.at[pl.ds(base, rows_per_w)],
                out_hbm.at[b, pl.ds(base, rows_per_w)],
                sem)
            cp.start()
            cps.append(cp)
        for cp in cps:
            cp.wait()

    return body(pos_emb)


# speedup vs baseline: 42.4706x; 42.4706x over previous
import functools

import jax
import jax.numpy as jnp
from jax import lax
from jax.experimental import pallas as pl
from jax.experimental.pallas import tpu as pltpu
from jax.experimental.pallas import tpu_sc as plsc

NBUF = 4
CHUNK = 16


def kernel(x, pos_emb):
    B, S = x.shape
    D = pos_emb.shape[1]

    info = plsc.get_sparse_core_info()
    NC, NS = info.num_cores, info.num_subcores
    NW = NC * NS
    rows_per_w = S // NW
    n_chunks = rows_per_w // CHUNK

    mesh = plsc.VectorSubcoreMesh(core_axis_name="c", subcore_axis_name="s")

    @functools.partial(
        pl.kernel,
        out_type=jax.ShapeDtypeStruct((B, S, D), jnp.float32),
        mesh=mesh,
        scratch_types=(
            [pltpu.VMEM((CHUNK, D), jnp.float32) for _ in range(NBUF)]
            + [pltpu.SemaphoreType.DMA for _ in range(2 * NBUF)]
        ),
    )
    def body(pos_hbm, out_hbm, *scratch):
        bufs = scratch[:NBUF]
        rsems = scratch[NBUF:2 * NBUF]
        wsems = scratch[2 * NBUF:]
        wid = lax.axis_index("s") * NC + lax.axis_index("c")
        base = wid * rows_per_w

        reads = [None] * n_chunks
        writes = [[] for _ in range(n_chunks)]

        def start_read(c):
            off = base + c * CHUNK
            cp = pltpu.make_async_copy(
                pos_hbm.at[pl.ds(off, CHUNK)], bufs[c % NBUF], rsems[c % NBUF])
            cp.start()
            reads[c] = cp

        for c in range(min(NBUF - 1, n_chunks)):
            start_read(c)
        for c in range(n_chunks):
            reads[c].wait()
            if c >= 1:
                for cp in writes[c - 1]:
                    cp.wait()
            if c + NBUF - 1 < n_chunks:
                start_read(c + NBUF - 1)
            off = base + c * CHUNK
            for b in range(B):
                cp = pltpu.make_async_copy(
                    bufs[c % NBUF], out_hbm.at[b, pl.ds(off, CHUNK)],
                    wsems[c % NBUF])
                cp.start()
                writes[c].append(cp)
        for cp in writes[n_chunks - 1]:
            cp.wait()

    return body(pos_emb)


# speedup vs baseline: 44.8441x; 1.0559x over previous
import functools

import jax
import jax.numpy as jnp
from jax import lax
from jax.experimental import pallas as pl
from jax.experimental.pallas import tpu as pltpu
from jax.experimental.pallas import tpu_sc as plsc

BUF_ROWS = (64, 56)
CHUNKS = (64, 56, 8)


def kernel(x, pos_emb):
    B, S = x.shape
    D = pos_emb.shape[1]

    info = plsc.get_sparse_core_info()
    NC, NS = info.num_cores, info.num_subcores
    NW = NC * NS
    rows_per_w = S // NW
    assert rows_per_w == sum(CHUNKS)
    n_chunks = len(CHUNKS)
    starts = [sum(CHUNKS[:i]) for i in range(n_chunks)]

    mesh = plsc.VectorSubcoreMesh(core_axis_name="c", subcore_axis_name="s")

    @functools.partial(
        pl.kernel,
        out_type=jax.ShapeDtypeStruct((B, S, D), jnp.float32),
        mesh=mesh,
        scratch_types=[
            pltpu.VMEM((BUF_ROWS[0], D), jnp.float32),
            pltpu.VMEM((BUF_ROWS[1], D), jnp.float32),
            pltpu.SemaphoreType.DMA,
            pltpu.SemaphoreType.DMA,
            pltpu.SemaphoreType.DMA,
            pltpu.SemaphoreType.DMA,
        ],
    )
    def body(pos_hbm, out_hbm, buf0, buf1, rsem0, rsem1, wsem0, wsem1):
        bufs = (buf0, buf1)
        rsems = (rsem0, rsem1)
        wsems = (wsem0, wsem1)
        wid = lax.axis_index("s") * NC + lax.axis_index("c")
        base = wid * rows_per_w

        reads = [None] * n_chunks
        writes = [[] for _ in range(n_chunks)]

        def start_read(c):
            off = base + starts[c]
            cp = pltpu.make_async_copy(
                pos_hbm.at[pl.ds(off, CHUNKS[c])],
                bufs[c % 2].at[pl.ds(0, CHUNKS[c])],
                rsems[c % 2])
            cp.start()
            reads[c] = cp

        start_read(0)
        for c in range(n_chunks):
            reads[c].wait()
            if c >= 1:
                for cp in writes[c - 1]:
                    cp.wait()
            if c + 1 < n_chunks:
                start_read(c + 1)
            off = base + starts[c]
            for b in range(B):
                cp = pltpu.make_async_copy(
                    bufs[c % 2].at[pl.ds(0, CHUNKS[c])],
                    out_hbm.at[b, pl.ds(off, CHUNKS[c])],
                    wsems[c % 2])
                cp.start()
                writes[c].append(cp)
        for cp in writes[n_chunks - 1]:
            cp.wait()

    return body(pos_emb)
